# baseline (speedup 1.0000x reference)
import functools

import jax
import jax.numpy as jnp
from jax import lax
from jax.experimental import pallas as pl
from jax.experimental.pallas import tpu as pltpu
from jax.experimental.pallas import tpu_sc as plsc

_VOCAB = 100000
_EMBED = 64
_BATCH = 1024
_TILE_V = 4096
_NSTEPS = pl.cdiv(_VOCAB, _TILE_V)
_IDX_ROW = 128


@functools.lru_cache(maxsize=None)
def _build_gather():
    info = plsc.get_sparse_core_info()
    nw = info.num_cores * info.num_subcores
    b_per_w = _BATCH // nw
    n_el = b_per_w * _EMBED
    n_grp = n_el // _IDX_ROW
    d_per_grp = _IDX_ROW // b_per_w
    mesh = plsc.VectorSubcoreMesh(core_axis_name="c", subcore_axis_name="s")

    @functools.partial(
        pl.kernel,
        out_type=jax.ShapeDtypeStruct((_EMBED * _BATCH, 1), jnp.float32),
        mesh=mesh,
        scratch_types=[
            pltpu.VMEM((b_per_w,), jnp.int32),
            pltpu.VMEM((n_grp, _IDX_ROW), jnp.int32),
            pltpu.VMEM((n_el, 1), jnp.float32),
            pltpu.SemaphoreType.DMA,
            pltpu.SemaphoreType.DMA,
        ],
        compiler_params=pltpu.CompilerParams(use_tc_tiling_on_sc=False),
    )
    def gather(tbl_hbm, ids_hbm, out_hbm, ids_v, idx_v, rows_v, sem, osem):
        wid = lax.axis_index("s") * info.num_cores + lax.axis_index("c")
        base = wid * b_per_w
        pltpu.sync_copy(ids_hbm.at[pl.ds(base, b_per_w)], ids_v)
        for g in range(n_grp):
            for d_sub in range(d_per_grp):
                d = g * d_per_grp + d_sub
                for h in range(b_per_w // 16):
                    ids16 = ids_v[pl.ds(h * 16, 16)]
                    idx_v[g, pl.ds(d_sub * b_per_w + h * 16, 16)] = (
                        ids16 + d * _VOCAB
                    )
        gathers = [
            pltpu.async_copy(
                tbl_hbm.at[idx_v.at[g]],
                rows_v.at[pl.ds(g * _IDX_ROW, _IDX_ROW)],
                sem,
            )
            for g in range(n_grp)
        ]
        for cp in gathers:
            cp.start()
        for cp in gathers:
            cp.wait()
        outs = [
            pltpu.async_copy(
                rows_v.at[pl.ds(d * b_per_w, b_per_w)],
                out_hbm.at[pl.ds(d * _BATCH + base, b_per_w)],
                osem,
            )
            for d in range(_EMBED)
        ]
        for cp in outs:
            cp.start()
        for cp in outs:
            cp.wait()

    return gather


def _matmul_body(w_ref, x_ref, o_ref):
    o_ref[...] = lax.dot_general(
        w_ref[...],
        x_ref[...],
        dimension_numbers=(((0,), (0,)), ((), ())),
        preferred_element_type=jnp.float32,
    )


def _matmul_t(wt, emb_t):
    return pl.pallas_call(
        _matmul_body,
        grid=(_NSTEPS,),
        in_specs=[
            pl.BlockSpec((_EMBED, _TILE_V), lambda j: (0, j)),
            pl.BlockSpec((_EMBED, _BATCH), lambda j: (0, 0)),
        ],
        out_specs=pl.BlockSpec((_TILE_V, _BATCH), lambda j: (j, 0)),
        out_shape=jax.ShapeDtypeStruct((_VOCAB, _BATCH), jnp.float32),
        compiler_params=pltpu.CompilerParams(
            vmem_limit_bytes=110 * 1024 * 1024,
        ),
    )(wt, emb_t)


def kernel(input_ids, emb_table, linear_w):
    tbl_flat = emb_table.T.reshape(_EMBED * _VOCAB, 1)
    emb_flat = _build_gather()(tbl_flat, input_ids.astype(jnp.int32))
    emb_t = emb_flat.reshape(_EMBED, _BATCH)
    out_t = _matmul_t(linear_w.T, emb_t)
    return out_t.T


# speedup vs baseline: 44.8600x; 44.8600x over previous
import functools

import jax
import jax.numpy as jnp
from jax import lax
from jax.experimental import pallas as pl
from jax.experimental.pallas import tpu as pltpu
from jax.experimental.pallas import tpu_sc as plsc

_VOCAB = 100000
_EMBED = 64
_EMBED_PAD = 128
_BATCH = 1024
_TILE_V = 4096
_NSTEPS = pl.cdiv(_VOCAB, _TILE_V)


@functools.lru_cache(maxsize=None)
def _build_gather():
    info = plsc.get_sparse_core_info()
    nw = info.num_cores * info.num_subcores
    b_per_w = _BATCH // nw
    mesh = plsc.VectorSubcoreMesh(core_axis_name="c", subcore_axis_name="s")

    @functools.partial(
        pl.kernel,
        out_type=jax.ShapeDtypeStruct((_BATCH, _EMBED_PAD), jnp.float32),
        mesh=mesh,
        scratch_types=[
            pltpu.VMEM((b_per_w,), jnp.int32),
            pltpu.VMEM((b_per_w, _EMBED_PAD), jnp.float32),
            pltpu.SemaphoreType.DMA,
        ],
    )
    def gather(table_hbm, idx_hbm, out_hbm, idx_v, rows_v, sem):
        wid = lax.axis_index("s") * info.num_cores + lax.axis_index("c")
        base = wid * b_per_w
        pltpu.sync_copy(idx_hbm.at[pl.ds(base, b_per_w)], idx_v)
        pltpu.async_copy(table_hbm.at[idx_v], rows_v, sem).wait()
        pltpu.sync_copy(rows_v, out_hbm.at[pl.ds(base, b_per_w)])

    return gather


def _matmul_body(w_ref, x_ref, o_ref):
    o_ref[...] = lax.dot_general(
        w_ref[...],
        x_ref[:, 0:_EMBED],
        dimension_numbers=(((0,), (1,)), ((), ())),
        preferred_element_type=jnp.float32,
    )


def _matmul_t(wt, emb):
    return pl.pallas_call(
        _matmul_body,
        grid=(_NSTEPS,),
        in_specs=[
            pl.BlockSpec((_EMBED, _TILE_V), lambda j: (0, j)),
            pl.BlockSpec((_BATCH, _EMBED_PAD), lambda j: (0, 0)),
        ],
        out_specs=pl.BlockSpec((_TILE_V, _BATCH), lambda j: (j, 0)),
        out_shape=jax.ShapeDtypeStruct((_VOCAB, _BATCH), jnp.float32),
        compiler_params=pltpu.CompilerParams(
            vmem_limit_bytes=110 * 1024 * 1024,
        ),
    )(wt, emb)


def kernel(input_ids, emb_table, linear_w):
    tbl_pad = jnp.pad(emb_table, ((0, 0), (0, _EMBED_PAD - _EMBED)))
    emb = _build_gather()(tbl_pad, input_ids.astype(jnp.int32))
    out_t = _matmul_t(linear_w.T, emb)
    return out_t.T


# speedup vs baseline: 44.8980x; 1.0008x over previous
import functools

import jax
import jax.numpy as jnp
from jax import lax
from jax.experimental import pallas as pl
from jax.experimental.pallas import tpu as pltpu
from jax.experimental.pallas import tpu_sc as plsc

_VOCAB = 100000
_EMBED = 64
_EMBED_PAD = 128
_BATCH = 1024
_TILE_V = 4096
_NSTEPS = pl.cdiv(_VOCAB, _TILE_V)


@functools.lru_cache(maxsize=None)
def _build_gather():
    info = plsc.get_sparse_core_info()
    nw = info.num_cores * info.num_subcores
    b_per_w = _BATCH // nw
    mesh = plsc.VectorSubcoreMesh(core_axis_name="c", subcore_axis_name="s")

    @functools.partial(
        pl.kernel,
        out_type=jax.ShapeDtypeStruct((_BATCH, _EMBED_PAD), jnp.float32),
        mesh=mesh,
        scratch_types=[
            pltpu.VMEM((b_per_w,), jnp.int32),
            pltpu.VMEM((b_per_w, _EMBED_PAD), jnp.float32),
            pltpu.SemaphoreType.DMA,
        ],
        compiler_params=pltpu.CompilerParams(use_tc_tiling_on_sc=True),
    )
    def gather(table_hbm, idx_hbm, out_hbm, idx_v, rows_v, sem):
        wid = lax.axis_index("s") * info.num_cores + lax.axis_index("c")
        base = wid * b_per_w
        pltpu.sync_copy(idx_hbm.at[pl.ds(base, b_per_w)], idx_v)
        pltpu.async_copy(table_hbm.at[idx_v], rows_v, sem).wait()
        pltpu.sync_copy(rows_v, out_hbm.at[pl.ds(base, b_per_w)])

    return gather


def _matmul_body(w_ref, x_ref, o_ref):
    o_ref[...] = lax.dot_general(
        w_ref[...],
        x_ref[:, 0:_EMBED],
        dimension_numbers=(((0,), (1,)), ((), ())),
        preferred_element_type=jnp.float32,
    )


def _matmul_t(wt, emb):
    return pl.pallas_call(
        _matmul_body,
        grid=(_NSTEPS,),
        in_specs=[
            pl.BlockSpec((_EMBED, _TILE_V), lambda j: (0, j)),
            pl.BlockSpec((_BATCH, _EMBED_PAD), lambda j: (0, 0)),
        ],
        out_specs=pl.BlockSpec((_TILE_V, _BATCH), lambda j: (j, 0)),
        out_shape=jax.ShapeDtypeStruct((_VOCAB, _BATCH), jnp.float32),
        compiler_params=pltpu.CompilerParams(
            vmem_limit_bytes=110 * 1024 * 1024,
        ),
    )(wt, emb)


def kernel(input_ids, emb_table, linear_w):
    tbl_pad = jnp.pad(emb_table, ((0, 0), (0, _EMBED_PAD - _EMBED)))
    emb = _build_gather()(tbl_pad, input_ids.astype(jnp.int32))
    out_t = _matmul_t(linear_w.T, emb)
    return out_t.T
